# baseline (speedup 1.0000x reference)
import functools

import jax
import jax.numpy as jnp
from jax import lax
from jax.experimental import pallas as pl
from jax.experimental.pallas import tpu as pltpu
from jax.experimental.pallas import tpu_sc as plsc

EPS_ = 1e-07
MAX_NORM_ = 1 - 0.0001

NUM_WORKERS = 32
ROWS_PER_CHUNK = 16
GROUP = 128
D = 16
LANE = 128


def _project_chunk(gbuf, sbuf, m, seq_len):
    lane = lax.iota(jnp.int32, 16)
    lanes = lane + m * ROWS_PER_CHUNK

    def block(j, carry):
        jj = jnp.full((16,), j, jnp.int32)
        diags = []
        ssum = jnp.zeros((16,), jnp.float32)
        for c in range(D):
            d2 = (lane + c) & (D - 1)
            dg = plsc.load_gather(gbuf, [lane, jj, d2])
            diags.append(dg)
            ssum = ssum + dg * dg
        bits = lax.bitcast_convert_type(ssum, jnp.int32)
        y = lax.bitcast_convert_type(
            jnp.int32(0x5F3759DF) - (bits >> 1), jnp.float32)
        for _ in range(3):
            y = y * (1.5 - 0.5 * ssum * y * y)
        norm = ssum * y
        factor = jnp.where(norm >= MAX_NORM_, MAX_NORM_ * y,
                           jnp.ones((16,), jnp.float32))
        for c in range(D):
            d2 = (lane + c) & (D - 1)
            plsc.store_scatter(sbuf, [jj * 2 + (d2 >> 3), d2 & 7, lanes],
                               diags[c] * factor)
        return carry

    lax.fori_loop(0, seq_len, block, 0)


def _make_sc_kernel(n_idx, seq_len):
    idx_rows_per_worker = n_idx // NUM_WORKERS
    groups = idx_rows_per_worker // GROUP
    chunks_per_group = GROUP // ROWS_PER_CHUNK
    tiles_per_group = 2 * seq_len
    info = plsc.get_sparse_core_info()
    nc = info.num_cores
    mesh = plsc.VectorSubcoreMesh(core_axis_name="c", subcore_axis_name="s")
    gbuf_t = pltpu.VMEM((ROWS_PER_CHUNK, seq_len, D), jnp.float32)
    sbuf_t = pltpu.VMEM((tiles_per_group, 8, LANE), jnp.float32)

    @functools.partial(
        pl.kernel,
        mesh=mesh,
        out_type=jax.ShapeDtypeStruct((tiles_per_group * (n_idx // GROUP),
                                       8, LANE), jnp.float32),
        compiler_params=pltpu.CompilerParams(needs_layout_passes=False,
                                             use_tc_tiling_on_sc=False),
        scratch_types=[
            pltpu.VMEM((3, 32, LANE), jnp.int32),
            pltpu.VMEM((idx_rows_per_worker, seq_len), jnp.int32),
            gbuf_t,
            gbuf_t,
            sbuf_t,
            pltpu.SemaphoreType.DMA,
            pltpu.SemaphoreType.DMA,
            pltpu.SemaphoreType.DMA,
        ],
    )
    def sc_kernel(idx_hbm, emb_hbm, out_hbm, idx_sl, idx_v, ga, gb, sa,
                  gsem_a, gsem_b, ssem_a):
        wid = lax.axis_index("s") * nc + lax.axis_index("c")
        base = wid * idx_rows_per_worker
        tcols = idx_rows_per_worker // LANE
        for tr in range(3):
            pltpu.sync_copy(
                idx_hbm.at[pl.ds((tr * (n_idx // LANE) + tcols * wid) * 8,
                                 8 * tcols)],
                idx_sl.at[tr])
        lane = lax.iota(jnp.int32, 16)

        def repack(c, carry):
            l = (c & 7) * 16 + lane
            i_l = c * 16 + lane
            srow = (c >> 3) * 8
            for j in range(seq_len):
                tr, s = j // 8, j % 8
                v = plsc.load_gather(
                    idx_sl, [jnp.full((16,), tr, jnp.int32),
                             jnp.zeros((16,), jnp.int32) + (srow + s), l])
                plsc.store_scatter(idx_v, [i_l, jnp.full((16,), j, jnp.int32)],
                                   v)
            return carry

        lax.fori_loop(0, idx_rows_per_worker // 16, repack, 0)

        def start_gathers(c, buf, gsem):
            r0 = c * ROWS_PER_CHUNK
            for k in range(ROWS_PER_CHUNK):
                pltpu.make_async_copy(
                    emb_hbm.at[idx_v.at[r0 + k]], buf.at[k], gsem).start()

        def wait_gathers(c, buf, gsem):
            r0 = c * ROWS_PER_CHUNK
            for k in range(ROWS_PER_CHUNK):
                pltpu.make_async_copy(
                    emb_hbm.at[idx_v.at[r0 + k]], buf.at[k], gsem).wait()

        def store_group(g, buf, ssem, wait):
            blk0 = (base // GROUP) + g
            for t in range(tiles_per_group):
                cp = pltpu.make_async_copy(
                    buf.at[t], out_hbm.at[t * (n_idx // GROUP) + blk0], ssem)
                if wait:
                    cp.wait()
                else:
                    cp.start()

        start_gathers(0, ga, gsem_a)

        def chunkstep(c, gbuf, gsem, ngbuf, ngsem):
            wait_gathers(c, gbuf, gsem)

            @pl.when(c + 1 < groups * chunks_per_group)
            def _():
                start_gathers(c + 1, ngbuf, ngsem)

            _project_chunk(gbuf, sa, c % chunks_per_group, seq_len)

        def gstep(g, carry):
            @pl.when(g >= 1)
            def _():
                store_group(g - 1, sa, ssem_a, wait=True)

            for h in range(chunks_per_group // 2):
                c = g * chunks_per_group + 2 * h
                chunkstep(c, ga, gsem_a, gb, gsem_b)
                chunkstep(c + 1, gb, gsem_b, ga, gsem_a)
            store_group(g, sa, ssem_a, wait=False)
            return carry

        lax.fori_loop(0, groups, gstep, 0)
        store_group(groups - 1, sa, ssem_a, wait=True)

    return sc_kernel


def kernel(idx, emb):
    n_idx, seq_len = idx.shape
    sp = -seq_len % 8
    idx_p = jnp.pad(idx.astype(jnp.int32).T, ((0, sp), (0, 0)))
    idx_b = (idx_p.reshape((seq_len + sp) // 8, 8, n_idx // LANE, LANE)
             .transpose(0, 2, 1, 3)
             .reshape((seq_len + sp) * n_idx // LANE, LANE))
    out4 = _make_sc_kernel(n_idx, seq_len)(idx_b, emb)
    out5 = out4.reshape(seq_len, 2, n_idx // GROUP, 8, LANE)
    return out5.transpose(2, 4, 0, 1, 3).reshape(n_idx, seq_len, D)


# speedup vs baseline: 1.0028x; 1.0028x over previous
import functools

import jax
import jax.numpy as jnp
from jax import lax
from jax.experimental import pallas as pl
from jax.experimental.pallas import tpu as pltpu
from jax.experimental.pallas import tpu_sc as plsc

EPS_ = 1e-07
MAX_NORM_ = 1 - 0.0001

NUM_WORKERS = 32
ROWS_PER_CHUNK = 16
GROUP = 128
D = 16
LANE = 128


def _project_chunk(gbuf, sbuf, m, seq_len):
    lane = lax.iota(jnp.int32, 16)
    lanes = lane + m * ROWS_PER_CHUNK

    def block(j, carry):
        jj = jnp.full((16,), j, jnp.int32)
        diags = []
        ssum = jnp.zeros((16,), jnp.float32)
        for c in range(D):
            d2 = (lane + c) & (D - 1)
            dg = plsc.load_gather(gbuf, [lane, jj, d2])
            diags.append(dg)
            ssum = ssum + dg * dg
        bits = lax.bitcast_convert_type(ssum, jnp.int32)
        y = lax.bitcast_convert_type(
            jnp.int32(0x5F3759DF) - (bits >> 1), jnp.float32)
        for _ in range(3):
            y = y * (1.5 - 0.5 * ssum * y * y)
        norm = ssum * y
        factor = jnp.where(norm >= MAX_NORM_, MAX_NORM_ * y,
                           jnp.ones((16,), jnp.float32))
        for c in range(D):
            d2 = (lane + c) & (D - 1)
            plsc.store_scatter(sbuf, [jj * 2 + (d2 >> 3), d2 & 7, lanes],
                               diags[c] * factor)
        return carry

    lax.fori_loop(0, seq_len, block, 0)


def _make_sc_kernel(n_idx, seq_len):
    idx_rows_per_worker = n_idx // NUM_WORKERS
    groups = idx_rows_per_worker // GROUP
    chunks_per_group = GROUP // ROWS_PER_CHUNK
    tiles_per_group = 2 * seq_len
    info = plsc.get_sparse_core_info()
    nc = info.num_cores
    mesh = plsc.VectorSubcoreMesh(core_axis_name="c", subcore_axis_name="s")
    gbuf_t = pltpu.VMEM((ROWS_PER_CHUNK, seq_len, D), jnp.float32)
    sbuf_t = pltpu.VMEM((tiles_per_group, 8, LANE), jnp.float32)

    @functools.partial(
        pl.kernel,
        mesh=mesh,
        out_type=jax.ShapeDtypeStruct((tiles_per_group * (n_idx // GROUP),
                                       8, LANE), jnp.float32),
        compiler_params=pltpu.CompilerParams(needs_layout_passes=False,
                                             use_tc_tiling_on_sc=False),
        scratch_types=[
            pltpu.VMEM((seq_len, idx_rows_per_worker), jnp.int32),
            pltpu.VMEM((idx_rows_per_worker, seq_len), jnp.int32),
            gbuf_t,
            gbuf_t,
            sbuf_t,
            pltpu.SemaphoreType.DMA,
            pltpu.SemaphoreType.DMA,
            pltpu.SemaphoreType.DMA,
        ],
    )
    def sc_kernel(idx_hbm, emb_hbm, out_hbm, idx_tv, idx_v, ga, gb, sa,
                  gsem_a, gsem_b, ssem_a):
        wid = lax.axis_index("s") * nc + lax.axis_index("c")
        base = wid * idx_rows_per_worker
        pltpu.sync_copy(idx_hbm.at[:, pl.ds(base, idx_rows_per_worker)],
                        idx_tv)
        lane = lax.iota(jnp.int32, 16)

        def repack(c, carry):
            col = c * 16 + lane
            for j in range(seq_len):
                jj = jnp.full((16,), j, jnp.int32)
                v = plsc.load_gather(idx_tv, [jj, col])
                plsc.store_scatter(idx_v, [col, jj], v)
            return carry

        lax.fori_loop(0, idx_rows_per_worker // 16, repack, 0)

        def start_gathers(c, buf, gsem):
            r0 = c * ROWS_PER_CHUNK
            for k in range(ROWS_PER_CHUNK):
                pltpu.make_async_copy(
                    emb_hbm.at[idx_v.at[r0 + k]], buf.at[k], gsem).start()

        def wait_gathers(c, buf, gsem):
            r0 = c * ROWS_PER_CHUNK
            for k in range(ROWS_PER_CHUNK):
                pltpu.make_async_copy(
                    emb_hbm.at[idx_v.at[r0 + k]], buf.at[k], gsem).wait()

        def store_group(g, buf, ssem, wait):
            blk0 = (base // GROUP) + g
            for t in range(tiles_per_group):
                cp = pltpu.make_async_copy(
                    buf.at[t], out_hbm.at[t * (n_idx // GROUP) + blk0], ssem)
                if wait:
                    cp.wait()
                else:
                    cp.start()

        start_gathers(0, ga, gsem_a)

        def chunkstep(c, gbuf, gsem, ngbuf, ngsem):
            wait_gathers(c, gbuf, gsem)

            @pl.when(c + 1 < groups * chunks_per_group)
            def _():
                start_gathers(c + 1, ngbuf, ngsem)

            _project_chunk(gbuf, sa, c % chunks_per_group, seq_len)

        def gstep(g, carry):
            @pl.when(g >= 1)
            def _():
                store_group(g - 1, sa, ssem_a, wait=True)

            for h in range(chunks_per_group // 2):
                c = g * chunks_per_group + 2 * h
                chunkstep(c, ga, gsem_a, gb, gsem_b)
                chunkstep(c + 1, gb, gsem_b, ga, gsem_a)
            store_group(g, sa, ssem_a, wait=False)
            return carry

        lax.fori_loop(0, groups, gstep, 0)
        store_group(groups - 1, sa, ssem_a, wait=True)

    return sc_kernel


def kernel(idx, emb):
    n_idx, seq_len = idx.shape
    out4 = _make_sc_kernel(n_idx, seq_len)(idx.astype(jnp.int32).T, emb)
    out5 = out4.reshape(seq_len, 2, n_idx // GROUP, 8, LANE)
    return out5.transpose(2, 4, 0, 1, 3).reshape(n_idx, seq_len, D)
